# Initial kernel scaffold; baseline (speedup 1.0000x reference)
import jax
import jax.numpy as jnp
from jax.experimental import pallas as pl


def kernel(values, labels):
    raise NotImplementedError("write your pallas kernel here")


# speedup vs baseline: 3.7734x; 3.7734x over previous
import functools

import jax
import jax.numpy as jnp
from jax import lax
from jax.experimental import pallas as pl
from jax.experimental.pallas import tpu as pltpu
from jax.experimental.pallas import tpu_sc as plsc

N_ROWS = 320000
D = 128
N_SEG = 10000
NC = 2
NS = 16
L = 16
NW = NC * NS
ROWS_PER_TILE = N_ROWS // NW
CHUNK = 80
N_CHUNKS = ROWS_PER_TILE // CHUNK
N_SEG_PAD = 10240
SEG_PER_TILE = N_SEG_PAD // NS
ZROWS = 128

_mesh = plsc.VectorSubcoreMesh(
    core_axis_name="c", subcore_axis_name="s", num_cores=NC, num_subcores=NS
)


@functools.partial(
    pl.kernel,
    out_type=jax.ShapeDtypeStruct((NC * N_SEG_PAD, D), jnp.float32),
    mesh=_mesh,
    scratch_types=[
        pltpu.VMEM((CHUNK, D), jnp.float32),
        pltpu.VMEM((CHUNK,), jnp.int32),
        pltpu.VMEM((ZROWS, D), jnp.float32),
        pltpu.VMEM_SHARED((N_SEG_PAD, D), jnp.float32),
    ],
)
def _sc_partial(values_hbm, labels_hbm, out_hbm, vals_v, labs_v, zbuf, acc_sh):
    cid = lax.axis_index("c")
    sid = lax.axis_index("s")
    wid = cid * NS + sid

    zv = jnp.zeros((L,), jnp.float32)

    def zero_row(i, carry):
        for j in range(D // L):
            zbuf[i, pl.ds(j * L, L)] = zv
        return carry

    lax.fori_loop(0, ZROWS, zero_row, 0)
    for r in range(SEG_PER_TILE // ZROWS):
        pltpu.sync_copy(zbuf, acc_sh.at[pl.ds(sid * SEG_PER_TILE + r * ZROWS, ZROWS)])
    plsc.subcore_barrier()

    base0 = wid * ROWS_PER_TILE

    def chunk_body(k, carry):
        base = base0 + k * CHUNK
        pltpu.sync_copy(values_hbm.at[pl.ds(base, CHUNK)], vals_v)
        pltpu.sync_copy(labels_hbm.at[pl.ds(base, CHUNK)], labs_v)
        pltpu.sync_copy(vals_v, acc_sh.at[labs_v], add=True)
        return carry

    lax.fori_loop(0, N_CHUNKS, chunk_body, 0)
    plsc.subcore_barrier()

    pltpu.sync_copy(
        acc_sh.at[pl.ds(sid * SEG_PER_TILE, SEG_PER_TILE)],
        out_hbm.at[pl.ds(cid * N_SEG_PAD + sid * SEG_PER_TILE, SEG_PER_TILE)],
    )


def _add_body(a_ref, b_ref, o_ref):
    o_ref[...] = a_ref[...] + b_ref[...]


_ADD_BLOCK = 1000


def _combine(partial):
    p3 = partial.reshape(NC, N_SEG_PAD, D)
    return pl.pallas_call(
        _add_body,
        grid=(N_SEG // _ADD_BLOCK,),
        in_specs=[
            pl.BlockSpec((None, _ADD_BLOCK, D), lambda i: (0, i, 0)),
            pl.BlockSpec((None, _ADD_BLOCK, D), lambda i: (1, i, 0)),
        ],
        out_specs=pl.BlockSpec((_ADD_BLOCK, D), lambda i: (i, 0)),
        out_shape=jax.ShapeDtypeStruct((N_SEG, D), jnp.float32),
    )(p3, p3)


def kernel(values, labels):
    labels32 = labels.astype(jnp.int32)
    partial = _sc_partial(values, labels32)
    return _combine(partial)


# speedup vs baseline: 7.4434x; 1.9726x over previous
import functools

import jax
import jax.numpy as jnp
from jax import lax
from jax.experimental import pallas as pl
from jax.experimental.pallas import tpu as pltpu
from jax.experimental.pallas import tpu_sc as plsc

N_ROWS = 320000
D = 128
N_SEG = 10000
NC = 2
NS = 16
L = 16
NW = NC * NS
ROWS_PER_TILE = N_ROWS // NW
CHUNK = 80
N_CHUNKS = ROWS_PER_TILE // CHUNK
N_SEG_PAD = 10240
SEG_PER_TILE = N_SEG_PAD // NS
ZROWS = 128
NBUF = 2

_mesh = plsc.VectorSubcoreMesh(
    core_axis_name="c", subcore_axis_name="s", num_cores=NC, num_subcores=NS
)


@functools.partial(
    pl.kernel,
    out_type=jax.ShapeDtypeStruct((NC * N_SEG_PAD, D), jnp.float32),
    mesh=_mesh,
    scratch_types=[
        pltpu.VMEM((NBUF, CHUNK, D), jnp.float32),
        pltpu.VMEM((NBUF, CHUNK), jnp.int32),
        pltpu.VMEM((ZROWS, D), jnp.float32),
        pltpu.VMEM_SHARED((N_SEG_PAD, D), jnp.float32),
        pltpu.SemaphoreType.DMA((NBUF,)),
    ],
)
def _sc_partial(values_hbm, labels_hbm, out_hbm, vals_v, labs_v, zbuf, acc_sh, sems):
    cid = lax.axis_index("c")
    sid = lax.axis_index("s")
    wid = cid * NS + sid

    zv = jnp.zeros((L,), jnp.float32)

    def zero_row(i, carry):
        for j in range(D // L):
            zbuf[i, pl.ds(j * L, L)] = zv
        return carry

    lax.fori_loop(0, ZROWS, zero_row, 0)
    for r in range(SEG_PER_TILE // ZROWS):
        pltpu.sync_copy(zbuf, acc_sh.at[pl.ds(sid * SEG_PER_TILE + r * ZROWS, ZROWS)])
    plsc.subcore_barrier()

    base0 = wid * ROWS_PER_TILE

    def start_load(k, b):
        base = base0 + k * CHUNK
        pltpu.async_copy(values_hbm.at[pl.ds(base, CHUNK)], vals_v.at[b], sems.at[b])
        pltpu.async_copy(labels_hbm.at[pl.ds(base, CHUNK)], labs_v.at[b], sems.at[b])

    def drain(b):
        pltpu.make_async_copy(values_hbm.at[pl.ds(0, CHUNK)], vals_v.at[b], sems.at[b]).wait()
        pltpu.make_async_copy(labels_hbm.at[pl.ds(0, CHUNK)], labs_v.at[b], sems.at[b]).wait()

    start_load(0, 0)

    @pl.loop(0, N_CHUNKS, step=NBUF)
    def _(k):
        for b in range(NBUF):
            kn = k + b + 1

            @pl.when(kn < N_CHUNKS)
            def _():
                start_load(kn, (b + 1) % NBUF)

            @pl.when(k + b < N_CHUNKS)
            def _():
                drain(b)
                pltpu.sync_copy(vals_v.at[b], acc_sh.at[labs_v.at[b]], add=True)

    plsc.subcore_barrier()
    pltpu.sync_copy(
        acc_sh.at[pl.ds(sid * SEG_PER_TILE, SEG_PER_TILE)],
        out_hbm.at[pl.ds(cid * N_SEG_PAD + sid * SEG_PER_TILE, SEG_PER_TILE)],
    )


def _add_body(a_ref, b_ref, o_ref):
    o_ref[...] = a_ref[...] + b_ref[...]


_ADD_BLOCK = 1000


def _combine(partial):
    p3 = partial.reshape(NC, N_SEG_PAD, D)
    return pl.pallas_call(
        _add_body,
        grid=(N_SEG // _ADD_BLOCK,),
        in_specs=[
            pl.BlockSpec((None, _ADD_BLOCK, D), lambda i: (0, i, 0)),
            pl.BlockSpec((None, _ADD_BLOCK, D), lambda i: (1, i, 0)),
        ],
        out_specs=pl.BlockSpec((_ADD_BLOCK, D), lambda i: (i, 0)),
        out_shape=jax.ShapeDtypeStruct((N_SEG, D), jnp.float32),
    )(p3, p3)


def kernel(values, labels):
    labels32 = labels.astype(jnp.int32)
    partial = _sc_partial(values, labels32)
    return _combine(partial)


# speedup vs baseline: 8.2546x; 1.1090x over previous
import functools

import jax
import jax.numpy as jnp
from jax import lax
from jax.experimental import pallas as pl
from jax.experimental.pallas import tpu as pltpu
from jax.experimental.pallas import tpu_sc as plsc

N_ROWS = 320000
D = 128
N_SEG = 10000
NC = 2
NS = 16
L = 16
NW = NC * NS
ROWS_PER_TILE = N_ROWS // NW
CHUNK = 80
N_CHUNKS = ROWS_PER_TILE // CHUNK
N_SEG_PAD = 10240
SEG_PER_TILE = N_SEG_PAD // NS
ZROWS = 128
NBUF = 4

_mesh = plsc.VectorSubcoreMesh(
    core_axis_name="c", subcore_axis_name="s", num_cores=NC, num_subcores=NS
)


@functools.partial(
    pl.kernel,
    out_type=jax.ShapeDtypeStruct((NC * N_SEG_PAD, D), jnp.float32),
    mesh=_mesh,
    scratch_types=[
        pltpu.VMEM((NBUF, CHUNK, D), jnp.float32),
        pltpu.VMEM((NBUF, CHUNK), jnp.int32),
        pltpu.VMEM_SHARED((N_SEG_PAD, D), jnp.float32),
        pltpu.SemaphoreType.DMA((NBUF,)),
        pltpu.SemaphoreType.DMA((NBUF,)),
    ],
)
def _sc_partial(values_hbm, labels_hbm, out_hbm, vals_v, labs_v, acc_sh,
                sem_ld, sem_sc):
    cid = lax.axis_index("c")
    sid = lax.axis_index("s")
    wid = cid * NS + sid

    zv = jnp.zeros((L,), jnp.float32)

    def zero_row(i, carry):
        for j in range(D // L):
            vals_v[0, i, pl.ds(j * L, L)] = zv
        return carry

    lax.fori_loop(0, CHUNK, zero_row, 0)
    for r in range(SEG_PER_TILE // CHUNK):
        pltpu.sync_copy(vals_v.at[0], acc_sh.at[pl.ds(sid * SEG_PER_TILE + r * CHUNK, CHUNK)])
    plsc.subcore_barrier()

    base0 = wid * ROWS_PER_TILE

    def start_load(c, b):
        base = base0 + c * CHUNK
        pltpu.async_copy(values_hbm.at[pl.ds(base, CHUNK)], vals_v.at[b], sem_ld.at[b])
        pltpu.async_copy(labels_hbm.at[pl.ds(base, CHUNK)], labs_v.at[b], sem_ld.at[b])

    def drain_load(b):
        pltpu.make_async_copy(values_hbm.at[pl.ds(0, CHUNK)], vals_v.at[b], sem_ld.at[b]).wait()
        pltpu.make_async_copy(labels_hbm.at[pl.ds(0, CHUNK)], labs_v.at[b], sem_ld.at[b]).wait()

    def start_scat(b):
        pltpu.async_copy(vals_v.at[b], acc_sh.at[labs_v.at[b]], sem_sc.at[b], add=True)

    def drain_scat(b):
        pltpu.make_async_copy(vals_v.at[b], acc_sh.at[labs_v.at[b]], sem_sc.at[b]).wait()

    for b in range(NBUF):
        start_load(b, b)

    @pl.loop(0, N_CHUNKS, step=NBUF)
    def _(k):
        for b in range(NBUF):
            c = k + b

            @pl.when(c < N_CHUNKS)
            def _():
                drain_load(b)
                start_scat(b)

            sp = (b + NBUF - 1) % NBUF
            cn = c - 1 + NBUF

            @pl.when(jnp.logical_and(c >= 1, cn < N_CHUNKS))
            def _():
                drain_scat(sp)
                start_load(cn, sp)

    for b in range(NBUF):
        drain_scat(b)

    plsc.subcore_barrier()
    pltpu.sync_copy(
        acc_sh.at[pl.ds(sid * SEG_PER_TILE, SEG_PER_TILE)],
        out_hbm.at[pl.ds(cid * N_SEG_PAD + sid * SEG_PER_TILE, SEG_PER_TILE)],
    )


def _add_body(a_ref, b_ref, o_ref):
    o_ref[...] = a_ref[...] + b_ref[...]


_ADD_BLOCK = 1000


def _combine(partial):
    p3 = partial.reshape(NC, N_SEG_PAD, D)
    return pl.pallas_call(
        _add_body,
        grid=(N_SEG // _ADD_BLOCK,),
        in_specs=[
            pl.BlockSpec((None, _ADD_BLOCK, D), lambda i: (0, i, 0)),
            pl.BlockSpec((None, _ADD_BLOCK, D), lambda i: (1, i, 0)),
        ],
        out_specs=pl.BlockSpec((_ADD_BLOCK, D), lambda i: (i, 0)),
        out_shape=jax.ShapeDtypeStruct((N_SEG, D), jnp.float32),
    )(p3, p3)


def kernel(values, labels):
    labels32 = labels.astype(jnp.int32)
    partial = _sc_partial(values, labels32)
    return _combine(partial)
